# baseline (speedup 1.0000x reference)
import functools

import jax
import jax.numpy as jnp
from jax import lax
from jax.experimental import pallas as pl
from jax.experimental.pallas import tpu as pltpu
from jax.experimental.pallas import tpu_sc as plsc

N = 50000
E = 800000
HID = 64
G = 128

NP = 50176
EROWS = 6416
EP = EROWS * 128
PAIRS = 25
NBUF = 6
HALF = 25000
ACC_ROWS = 25088
TRASH = 25080
TPS = 1568


def _sc_aggr(d_feat, n_tables):
    mesh = plsc.VectorSubcoreMesh(core_axis_name="c", subcore_axis_name="s")

    @functools.partial(
        pl.kernel,
        out_type=[jax.ShapeDtypeStruct((NP, d_feat), jnp.float32)] * n_tables,
        mesh=mesh,
        compiler_params=pltpu.CompilerParams(use_tc_tiling_on_sc=False),
        scratch_types=(
            [
                pltpu.VMEM((2, 16, 128), jnp.int32),
                pltpu.VMEM((2, 16, 128), jnp.int32),
                pltpu.VMEM((2, 16, 128), jnp.int32),
                pltpu.VMEM_SHARED((ACC_ROWS, d_feat), jnp.float32),
            ]
            + [pltpu.VMEM((128, d_feat), jnp.float32)] * NBUF
            + [pltpu.SemaphoreType.DMA] * (2 * NBUF + 4)
        ),
    )
    def k(src_hbm, dst_hbm, zeros_hbm, *rest):
        tables = rest[:n_tables]
        outs = rest[n_tables:2 * n_tables]
        srcb, dstb, idxb, acc = rest[2 * n_tables:2 * n_tables + 4]
        rest = rest[2 * n_tables + 4:]
        bufs = rest[:NBUF]
        gsem = rest[NBUF:2 * NBUF]
        ssem = rest[2 * NBUF:3 * NBUF]
        stg_src = rest[3 * NBUF:3 * NBUF + 2]
        stg_dst = rest[3 * NBUF + 2:3 * NBUF + 4]
        c = lax.axis_index("c")
        s = lax.axis_index("s")
        lo = c * HALF

        def stage_first():
            pltpu.async_copy(src_hbm.at[pl.ds(s * 400, 16)], srcb.at[0],
                             stg_src[0])
            pltpu.async_copy(dst_hbm.at[pl.ds(s * 400, 16)], dstb.at[0],
                             stg_dst[0])

        pltpu.sync_copy(zeros_hbm, acc.at[pl.ds(s * TPS, TPS)])
        stage_first()
        plsc.subcore_barrier()

        for ti in range(n_tables):
            vals_hbm = tables[ti]

            def process(p, i):
                q = 1 - p
                nr0 = s * 400 + (i + 1) * 16
                pltpu.async_copy(src_hbm.at[pl.ds(nr0, 16)], srcb.at[q],
                                 stg_src[q])
                pltpu.async_copy(dst_hbm.at[pl.ds(nr0, 16)], dstb.at[q],
                                 stg_dst[q])
                pltpu.make_async_copy(src_hbm.at[pl.ds(0, 16)], srcb.at[p],
                                      stg_src[p]).wait()
                pltpu.make_async_copy(dst_hbm.at[pl.ds(0, 16)], dstb.at[p],
                                      stg_dst[p]).wait()
                gd, sd = {}, {}
                for j in range(3):
                    gd[j] = pltpu.async_copy(vals_hbm.at[srcb.at[p, j]],
                                             bufs[j % NBUF], gsem[j % NBUF])
                for r in range(16):
                    for t in range(8):
                        d = dstb[p, r, pl.ds(t * 16, 16)]
                        dl = d - lo
                        ok = (dl >= 0) & (dl < HALF)
                        idxb[p, r, pl.ds(t * 16, 16)] = (
                            jnp.where(ok, dl, TRASH))
                for j in range(16):
                    gd[j].wait()
                    sd[j] = pltpu.async_copy(bufs[j % NBUF],
                                             acc.at[idxb.at[p, j]],
                                             ssem[j % NBUF], add=True)
                    nj = j + 3
                    if nj < 16:
                        if nj - NBUF >= 0:
                            sd[nj - NBUF].wait()
                        gd[nj] = pltpu.async_copy(
                            vals_hbm.at[srcb.at[p, nj]],
                            bufs[nj % NBUF], gsem[nj % NBUF])
                for j in range(16 - NBUF, 16):
                    sd[j].wait()

            def body(i, carry):
                @pl.when(i % 2 == 0)
                def _():
                    process(0, i)

                @pl.when(i % 2 == 1)
                def _():
                    process(1, i)

                return carry

            lax.fori_loop(0, PAIRS, body, 0)
            pltpu.make_async_copy(src_hbm.at[pl.ds(0, 16)], srcb.at[1],
                                  stg_src[1]).wait()
            pltpu.make_async_copy(dst_hbm.at[pl.ds(0, 16)], dstb.at[1],
                                  stg_dst[1]).wait()
            plsc.subcore_barrier()
            out_hbm = outs[ti]

            @pl.when(s < 15)
            def _():
                pltpu.sync_copy(acc.at[pl.ds(s * TPS, TPS)],
                                out_hbm.at[pl.ds(lo + s * TPS, TPS)])

            @pl.when(s == 15)
            def _():
                pltpu.sync_copy(acc.at[pl.ds(15 * TPS, HALF - 15 * TPS)],
                                out_hbm.at[pl.ds(lo + 15 * TPS,
                                                 HALF - 15 * TPS)])

            if ti + 1 < n_tables:
                pltpu.sync_copy(zeros_hbm, acc.at[pl.ds(s * TPS, TPS)])
                stage_first()
                plsc.subcore_barrier()

    return k


_sc_aggr16 = _sc_aggr(16, 1)
_sc_aggr32x2 = _sc_aggr(32, 2)


BM = 1568


def _tc1_body(x_ref, a_ref, w1_ref, b1_ref, w2_ref, b2_ref,
              o_ref, oa_ref, ob_ref):
    z = x_ref[...] + a_ref[...]
    h = jnp.maximum(
        jnp.dot(z, w1_ref[...], preferred_element_type=jnp.float32)
        + b1_ref[...], 0.0)
    h1 = jnp.maximum(
        jnp.dot(h, w2_ref[...], preferred_element_type=jnp.float32)
        + b2_ref[...], 0.0)
    o_ref[...] = h1
    oa_ref[...] = h1[:, :HID // 2]
    ob_ref[...] = h1[:, HID // 2:]


def _tc1(x_pad, aggr1, W1p, b1, W2, b2):
    grid = NP // BM
    return pl.pallas_call(
        _tc1_body,
        grid=(grid,),
        in_specs=[
            pl.BlockSpec((BM, 16), lambda i: (i, 0)),
            pl.BlockSpec((BM, 16), lambda i: (i, 0)),
            pl.BlockSpec((16, HID), lambda i: (0, 0)),
            pl.BlockSpec((1, HID), lambda i: (0, 0)),
            pl.BlockSpec((HID, HID), lambda i: (0, 0)),
            pl.BlockSpec((1, HID), lambda i: (0, 0)),
        ],
        out_specs=[
            pl.BlockSpec((BM, HID), lambda i: (i, 0)),
            pl.BlockSpec((BM, HID // 2), lambda i: (i, 0)),
            pl.BlockSpec((BM, HID // 2), lambda i: (i, 0)),
        ],
        out_shape=[
            jax.ShapeDtypeStruct((NP, HID), jnp.float32),
            jax.ShapeDtypeStruct((NP, HID // 2), jnp.float32),
            jax.ShapeDtypeStruct((NP, HID // 2), jnp.float32),
        ],
        compiler_params=pltpu.CompilerParams(
            dimension_semantics=("arbitrary",)),
    )(x_pad, aggr1, W1p, b1, W2, b2)


def _tc2_body(h_ref, aa_ref, ab_ref, b_ref, w3_ref, b3_ref, w4_ref, b4_ref,
              pool_ref, out_ref):
    i = pl.program_id(0)
    nblk = pl.num_programs(0)
    a = jnp.concatenate([aa_ref[...], ab_ref[...]], axis=1)
    z = h_ref[...] + a
    t = jnp.maximum(
        jnp.dot(z, w3_ref[...], preferred_element_type=jnp.float32)
        + b3_ref[...], 0.0)
    h2 = jnp.maximum(
        jnp.dot(t, w4_ref[...], preferred_element_type=jnp.float32)
        + b4_ref[...], 0.0)
    bidx = b_ref[...]
    valid = bidx < G
    h2 = jnp.where(valid, h2, 0.0)
    onehot = (bidx == lax.broadcasted_iota(jnp.int32, (BM, G), 1))
    onehot = onehot.astype(jnp.float32)
    ones_col = jnp.where(valid, 1.0, 0.0)
    hc = jnp.concatenate(
        [h2, ones_col, jnp.zeros((BM, 15), jnp.float32)], axis=1)
    contrib = lax.dot_general(
        onehot, hc, (((0,), (0,)), ((), ())),
        preferred_element_type=jnp.float32)

    @pl.when(i == 0)
    def _():
        pool_ref[...] = jnp.zeros_like(pool_ref)

    pool_ref[...] += contrib

    @pl.when(i == nblk - 1)
    def _():
        p = pool_ref[...]
        cnt = jnp.maximum(p[:, HID:HID + 1], 1.0)
        out_ref[...] = p[:, :HID] / cnt


def _tc2(h1, aggr2a, aggr2b, batch2d, W3, b3, W4, b4):
    grid = NP // BM
    _, out = pl.pallas_call(
        _tc2_body,
        grid=(grid,),
        in_specs=[
            pl.BlockSpec((BM, HID), lambda i: (i, 0)),
            pl.BlockSpec((BM, HID // 2), lambda i: (i, 0)),
            pl.BlockSpec((BM, HID // 2), lambda i: (i, 0)),
            pl.BlockSpec((BM, 1), lambda i: (i, 0)),
            pl.BlockSpec((HID, HID), lambda i: (0, 0)),
            pl.BlockSpec((1, HID), lambda i: (0, 0)),
            pl.BlockSpec((HID, HID), lambda i: (0, 0)),
            pl.BlockSpec((1, HID), lambda i: (0, 0)),
        ],
        out_specs=[
            pl.BlockSpec((G, HID + 16), lambda i: (0, 0)),
            pl.BlockSpec((G, HID), lambda i: (0, 0)),
        ],
        out_shape=[
            jax.ShapeDtypeStruct((G, HID + 16), jnp.float32),
            jax.ShapeDtypeStruct((G, HID), jnp.float32),
        ],
        compiler_params=pltpu.CompilerParams(
            dimension_semantics=("arbitrary",)),
    )(h1, aggr2a, aggr2b, batch2d, W3, b3, W4, b4)
    return out


def kernel(x, edge_index, batch, W1, b1, W2, b2, W3, b3, W4, b4):
    ei = jnp.asarray(edge_index, jnp.int32)
    src = jnp.concatenate([ei[0], jnp.zeros((EP - E,), jnp.int32)])
    dst = jnp.concatenate([ei[1], jnp.full((EP - E,), 1 << 20, jnp.int32)])
    src2d = src.reshape(EROWS, 128)
    dst2d = dst.reshape(EROWS, 128)

    x_pad = jnp.pad(x, ((0, NP - N), (0, 16 - x.shape[1])))
    W1p = jnp.pad(W1, ((0, 16 - W1.shape[0]), (0, 0)))
    batch2d = jnp.concatenate(
        [jnp.asarray(batch, jnp.int32),
         jnp.full((NP - N,), G, jnp.int32)]).reshape(NP, 1)

    z16 = jnp.zeros((TPS, 16), jnp.float32)
    z32 = jnp.zeros((TPS, HID // 2), jnp.float32)

    (aggr1,) = _sc_aggr16(src2d, dst2d, z16, x_pad)
    h1, h1a, h1b = _tc1(x_pad, aggr1, W1p, b1.reshape(1, HID),
                        W2, b2.reshape(1, HID))
    aggr2a, aggr2b = _sc_aggr32x2(src2d, dst2d, z32, h1a, h1b)
    return _tc2(h1, aggr2a, aggr2b, batch2d, W3, b3.reshape(1, HID),
                W4, b4.reshape(1, HID))
